# Initial kernel scaffold; baseline (speedup 1.0000x reference)
import jax
import jax.numpy as jnp
from jax.experimental import pallas as pl


def kernel(x, feature_grid):
    raise NotImplementedError("write your pallas kernel here")


# speedup vs baseline: 2.8527x; 2.8527x over previous
import functools

import jax
import jax.numpy as jnp
from jax import lax
from jax.experimental import pallas as pl
from jax.experimental.pallas import tpu as pltpu
from jax.experimental.pallas import tpu_sc as plsc

C = 16
G = 128
NC = 2
NS = 16
NW = NC * NS
L = 16
B = 256
NIDX = 8 * B

_CORNERS = [(dz, dy, dx) for dz in (0, 1) for dy in (0, 1) for dx in (0, 1)]
_OFFS = [dz * G * G + dy * G + dx for (dz, dy, dx) in _CORNERS]


def _body(nchunks, npad, xT, table, out, coords, idxb, wb, rows, outb, sem):
    wid = lax.axis_index("s") * NC + lax.axis_index("c")
    my_chunks = (nchunks - 1 - wid) // NW + 1

    def chunk_body(i, carry):
        chunk = wid + i * NW
        off = chunk * B
        pltpu.sync_copy(xT.at[pl.ds(off, B)], coords.at[pl.ds(0, B)])
        pltpu.sync_copy(xT.at[pl.ds(npad + off, B)], coords.at[pl.ds(B, B)])
        pltpu.sync_copy(xT.at[pl.ds(2 * npad + off, B)], coords.at[pl.ds(2 * B, B)])

        for g in range(B // L):
            s = g * L
            vx = coords[pl.ds(s, L)]
            vy = coords[pl.ds(B + s, L)]
            vz = coords[pl.ds(2 * B + s, L)]
            ix = (vx + 1.0) * (0.5 * (G - 1))
            iy = (vy + 1.0) * (0.5 * (G - 1))
            iz = (vz + 1.0) * (0.5 * (G - 1))
            x0 = jnp.minimum(jnp.maximum(ix.astype(jnp.int32), 0), G - 2)
            y0 = jnp.minimum(jnp.maximum(iy.astype(jnp.int32), 0), G - 2)
            z0 = jnp.minimum(jnp.maximum(iz.astype(jnp.int32), 0), G - 2)
            fx = ix - x0.astype(jnp.float32)
            fy = iy - y0.astype(jnp.float32)
            fz = iz - z0.astype(jnp.float32)
            gx = 1.0 - fx
            gy = 1.0 - fy
            gz = 1.0 - fz
            base = z0 * (G * G) + y0 * G + x0
            wzy = [gz * gy, gz * fy, fz * gy, fz * fy]
            wx = [gx, fx]
            for c in range(8):
                pos = c * B + s
                idxb[pos // 128, pl.ds(pos % 128, L)] = base + _OFFS[c]
                wb[pl.ds(pos, L)] = wzy[c >> 1] * wx[c & 1]

        cps = [
            pltpu.async_copy(
                table.at[idxb.at[j]], rows.at[pl.ds(j * 128, 128)], sem
            )
            for j in range(NIDX // 128)
        ]
        for cp in cps:
            cp.wait()

        def grp_body(g, c2):
            s = g * L
            wvs = [wb[pl.ds(c * B + s, L)] for c in range(8)]
            for lane in range(L):
                p = s + lane
                acc = wvs[0][lane] * rows[p]
                for c in range(1, 8):
                    acc = acc + wvs[c][lane] * rows[c * B + p]
                outb[p] = acc
            return c2

        lax.fori_loop(0, B // L, grp_body, 0)
        pltpu.sync_copy(outb, out.at[pl.ds(off, B)])
        return carry

    lax.fori_loop(0, my_chunks, chunk_body, 0)


def kernel(x, feature_grid):
    n = x.shape[0]
    nchunks = (n + B - 1) // B
    n_pad = nchunks * B
    grid = feature_grid[0]
    table = jnp.transpose(grid, (1, 2, 3, 0)).reshape(G * G * G, C)
    xT = jnp.pad(x, ((0, n_pad - n), (0, 0))).T.reshape(-1)

    mesh = plsc.VectorSubcoreMesh(core_axis_name="c", subcore_axis_name="s")
    run = pl.kernel(
        functools.partial(_body, nchunks, n_pad),
        out_type=jax.ShapeDtypeStruct((n_pad, C), jnp.float32),
        mesh=mesh,
        compiler_params=pltpu.CompilerParams(use_tc_tiling_on_sc=False),
        scratch_types=[
            pltpu.VMEM((3 * B,), jnp.float32),
            pltpu.VMEM((NIDX // 128, 128), jnp.int32),
            pltpu.VMEM((NIDX,), jnp.float32),
            pltpu.VMEM((NIDX, C), jnp.float32),
            pltpu.VMEM((B, C), jnp.float32),
            pltpu.SemaphoreType.DMA,
        ],
    )
    outp = run(xT, table)
    return outp[:n]
